# baseline (speedup 1.0000x reference)
import jax
import jax.numpy as jnp
from jax.experimental import pallas as pl

_BANK = 100000
_DIM = 128
_BATCH = 16384
_BLK = 16384
_NB = (_BANK + _BLK - 1) // _BLK
_SPLIT = _BATCH // _BLK


def _body(feat_ref, bank_ref, lab_ref, lbank_ref, out_fb_ref, out_lb_ref):
    i = pl.program_id(0)

    @pl.when(i < _SPLIT)
    def _():
        out_fb_ref[...] = feat_ref[...]

    @pl.when(i >= _SPLIT)
    def _():
        out_fb_ref[...] = bank_ref[...]

    @pl.when(i == 0)
    def _():
        out_lb_ref[0:_BATCH] = lab_ref[...]
        out_lb_ref[_BATCH:_BANK] = lbank_ref[_BATCH:_BANK]


def kernel(features, labels, feature_bank, label_bank):
    out_fb, out_lb = pl.pallas_call(
        _body,
        grid=(_NB,),
        in_specs=[
            pl.BlockSpec((_BLK, _DIM), lambda i: (jnp.minimum(i, _SPLIT - 1), 0)),
            pl.BlockSpec((_BLK, _DIM), lambda i: (jnp.maximum(i, _SPLIT), 0)),
            pl.BlockSpec((_BATCH,), lambda i: (0,)),
            pl.BlockSpec((_BANK,), lambda i: (0,)),
        ],
        out_specs=[
            pl.BlockSpec((_BLK, _DIM), lambda i: (i, 0)),
            pl.BlockSpec((_BANK,), lambda i: (0,)),
        ],
        out_shape=[
            jax.ShapeDtypeStruct((_BANK, _DIM), feature_bank.dtype),
            jax.ShapeDtypeStruct((_BANK,), label_bank.dtype),
        ],
    )(features, feature_bank, labels, label_bank)
    return out_fb, out_lb
